# baseline (speedup 1.0000x reference)
import functools

import jax
import jax.numpy as jnp
from jax import lax
from jax.experimental import pallas as pl
from jax.experimental.pallas import tpu as pltpu
from jax.experimental.pallas import tpu_sc as plsc

KNN = 16
NPTS = 10000
CH = 64
NPAD = 10240
RB = 256
NSTEP = NPAD // RB

NWORK = 32
PTS_PER_W = NPAD // NWORK
CHUNK_PTS = 64
NCHUNK = PTS_PER_W // CHUNK_PTS
CHUNK_ROWS = CHUNK_PTS * KNN
IDX_SB = CHUNK_ROWS // 128

CB1 = 400
CB2 = 512


def _topk_body(xp_ref, xtp_ref, w1t_ref, wdt_ref, idx_ref, ut_ref, wt_ref,
               xx_ref):
    i = pl.program_id(0)

    @pl.when(i == 0)
    def _():
        xpv = xp_ref[...]
        xx = jnp.sum(xpv * xpv, axis=0, keepdims=True)
        lane = lax.broadcasted_iota(jnp.int32, (1, NPAD), 1)
        xx_ref[...] = jnp.where(lane >= NPTS, jnp.float32(jnp.inf), xx)

    xt = xtp_ref[...]
    dot = jnp.dot(xt, xp_ref[...], preferred_element_type=jnp.float32)
    s0 = 2.0 * dot - xx_ref[...]
    lane16 = lax.broadcasted_iota(jnp.int32, (RB, KNN), 1)
    colf = lax.broadcasted_iota(jnp.int32, (NPAD, 1), 0).astype(jnp.float32)
    neg = jnp.float32(-jnp.inf)

    m0 = jnp.max(s0, axis=1, keepdims=True)

    def kbody(k, carry):
        m, acc = carry
        hitf = jnp.where(s0 == m, 1.0, 0.0)
        jf = jnp.dot(hitf, colf, preferred_element_type=jnp.float32)
        j = jnp.clip(jf.astype(jnp.int32), 0, NPTS - 1)
        acc = jnp.where(lane16 == k, j, acc)
        m = jnp.max(jnp.where(s0 < m, s0, neg), axis=1, keepdims=True)
        return m, acc

    _, idxacc = lax.fori_loop(
        0, KNN, kbody, (m0, jnp.zeros((RB, KNN), jnp.int32)), unroll=4)
    idx_ref[...] = idxacc
    ut_ref[...] = jnp.dot(xt, w1t_ref[...], preferred_element_type=jnp.float32)
    wt_ref[...] = jnp.dot(xt, wdt_ref[...], preferred_element_type=jnp.float32)


_topk_call = pl.pallas_call(
    _topk_body,
    grid=(NSTEP,),
    in_specs=[
        pl.BlockSpec((CH, NPAD), lambda i: (0, 0)),
        pl.BlockSpec((RB, CH), lambda i: (i, 0)),
        pl.BlockSpec((CH, CH), lambda i: (0, 0)),
        pl.BlockSpec((CH, CH), lambda i: (0, 0)),
    ],
    out_specs=[
        pl.BlockSpec((RB, KNN), lambda i: (i, 0)),
        pl.BlockSpec((RB, CH), lambda i: (i, 0)),
        pl.BlockSpec((RB, CH), lambda i: (i, 0)),
    ],
    out_shape=[
        jax.ShapeDtypeStruct((NPAD, KNN), jnp.int32),
        jax.ShapeDtypeStruct((NPAD, CH), jnp.float32),
        jax.ShapeDtypeStruct((NPAD, CH), jnp.float32),
    ],
    scratch_shapes=[pltpu.VMEM((1, NPAD), jnp.float32)],
)


def _sc_body(ut_hbm, idx_hbm, s1_hbm, s2_hbm, m_hbm,
             idx_v, g_v, s1_v, s2_v, m_v, sem):
    wid = lax.axis_index("s") * 2 + lax.axis_index("c")

    def chunk_body(ci, carry):
        rowbase = wid * (PTS_PER_W * KNN // 128) + ci * IDX_SB
        pltpu.sync_copy(idx_hbm.at[pl.ds(rowbase, IDX_SB)], idx_v)
        for h in range(2):
            cps = [
                pltpu.async_copy(ut_hbm.at[idx_v.at[h * 4 + sb]],
                                 g_v.at[pl.ds(sb * 128, 128)], sem)
                for sb in range(4)
            ]
            for cp in cps:
                cp.wait()

            def pt_body(p, c2):
                r0 = p * KNN
                for c in range(4):
                    cs = pl.ds(c * 16, 16)
                    v = g_v[r0, cs]
                    s1 = v
                    s2 = v * v
                    mx = v
                    for r in range(1, KNN):
                        vv = g_v[r0 + r, cs]
                        s1 = s1 + vv
                        s2 = s2 + vv * vv
                        mx = jnp.maximum(mx, vv)
                    s1_v[p, cs] = s1
                    s2_v[p, cs] = s2
                    m_v[p, cs] = mx
                return c2

            lax.fori_loop(0, CHUNK_PTS // 2, pt_body, 0)
            pt0 = wid * PTS_PER_W + ci * CHUNK_PTS + h * (CHUNK_PTS // 2)
            pltpu.sync_copy(s1_v, s1_hbm.at[pl.ds(pt0, CHUNK_PTS // 2)])
            pltpu.sync_copy(s2_v, s2_hbm.at[pl.ds(pt0, CHUNK_PTS // 2)])
            pltpu.sync_copy(m_v, m_hbm.at[pl.ds(pt0, CHUNK_PTS // 2)])
        return carry

    lax.fori_loop(0, NCHUNK, chunk_body, 0)


@functools.lru_cache(maxsize=1)
def _get_sc_call():
    return functools.partial(
        pl.kernel,
        mesh=plsc.VectorSubcoreMesh(core_axis_name="c", subcore_axis_name="s"),
        out_type=[jax.ShapeDtypeStruct((NPAD, CH), jnp.float32)] * 3,
        scratch_types=[
            pltpu.VMEM((IDX_SB, 128), jnp.int32),
            pltpu.VMEM((CHUNK_ROWS // 2, 128), jnp.float32),
            pltpu.VMEM((CHUNK_PTS // 2, CH), jnp.float32),
            pltpu.VMEM((CHUNK_PTS // 2, CH), jnp.float32),
            pltpu.VMEM((CHUNK_PTS // 2, CH), jnp.float32),
            pltpu.SemaphoreType.DMA,
        ],
    )(_sc_body)


def _c1_body(s1_ref, s2_ref, wt_ref, acc_ref):
    i = pl.program_id(0)
    s1 = s1_ref[...]
    s2 = s2_ref[...]
    wt = wt_ref[...]
    p = jnp.concatenate([
        jnp.sum(s1, axis=0, keepdims=True),
        jnp.sum(s2, axis=0, keepdims=True),
        jnp.sum(wt * s1, axis=0, keepdims=True),
        jnp.sum(wt, axis=0, keepdims=True),
        jnp.sum(wt * wt, axis=0, keepdims=True),
        jnp.zeros((3, CH), jnp.float32),
    ], axis=0)

    @pl.when(i == 0)
    def _():
        acc_ref[...] = p

    @pl.when(i > 0)
    def _():
        acc_ref[...] = acc_ref[...] + p


_c1_call = pl.pallas_call(
    _c1_body,
    grid=(NPTS // CB1,),
    in_specs=[
        pl.BlockSpec((CB1, CH), lambda i: (i, 0)),
        pl.BlockSpec((CB1, CH), lambda i: (i, 0)),
        pl.BlockSpec((CB1, CH), lambda i: (i, 0)),
    ],
    out_specs=pl.BlockSpec((8, CH), lambda i: (0, 0)),
    out_shape=jax.ShapeDtypeStruct((8, CH), jnp.float32),
)


def _c2_body(m_ref, wt_ref, sums_ref, gam_ref, bet_ref, out_ref):
    sums = sums_ref[...]
    cnt = jnp.float32(NPTS * KNN)
    mean = (sums[0:1] + KNN * sums[3:4]) / cnt
    ey2 = (sums[1:2] + 2.0 * sums[2:3] + KNN * sums[4:5]) / cnt
    var = ey2 - mean * mean
    rstd = lax.rsqrt(var + 1e-5)
    z = (m_ref[...] + wt_ref[...] - mean) * (rstd * gam_ref[...]) + bet_ref[...]
    act = jnp.where(z > 0, z, 0.2 * z)
    out_ref[...] = act.T


_c2_call = pl.pallas_call(
    _c2_body,
    grid=(NPAD // CB2,),
    in_specs=[
        pl.BlockSpec((CB2, CH), lambda i: (i, 0)),
        pl.BlockSpec((CB2, CH), lambda i: (i, 0)),
        pl.BlockSpec((8, CH), lambda i: (0, 0)),
        pl.BlockSpec((1, CH), lambda i: (0, 0)),
        pl.BlockSpec((1, CH), lambda i: (0, 0)),
    ],
    out_specs=pl.BlockSpec((CH, CB2), lambda i: (0, i)),
    out_shape=jax.ShapeDtypeStruct((CH, NPAD), jnp.float32),
)


def kernel(x, W, gamma, beta):
    _, c, n = x.shape
    x2 = x[0]
    xp = jnp.pad(x2, ((0, 0), (0, NPAD - n)))
    xtp = xp.T
    w1t = W[:, :c].T
    wdt = (W[:, c:] - W[:, :c]).T

    idx, ut, wt = _topk_call(xp, xtp, w1t, wdt)
    idx2 = idx.reshape(-1, 128)
    utp = jnp.pad(ut, ((0, 0), (0, 128 - CH)))
    s1, s2, m = _get_sc_call()(utp, idx2)

    sums = _c1_call(s1, s2, wt)
    out = _c2_call(m, wt, sums, gamma.reshape(1, -1), beta.reshape(1, -1))
    return out[None, :, :n]


# speedup vs baseline: 1.0001x; 1.0001x over previous
import functools

import jax
import jax.numpy as jnp
from jax import lax
from jax.experimental import pallas as pl
from jax.experimental.pallas import tpu as pltpu
from jax.experimental.pallas import tpu_sc as plsc

KNN = 16
NPTS = 10000
CH = 64
NPAD = 10240
RB = 256
NSTEP = NPAD // RB

NWORK = 32
PTS_PER_W = NPAD // NWORK
CHUNK_PTS = 64
NCHUNK = PTS_PER_W // CHUNK_PTS
CHUNK_ROWS = CHUNK_PTS * KNN
IDX_SB = CHUNK_ROWS // 128

CB1 = 400
CB2 = 512


def _topk_body(xp_ref, xtp_ref, w1t_ref, wdt_ref, idx_ref, ut_ref, wt_ref,
               xx_ref):
    i = pl.program_id(0)

    @pl.when(i == 0)
    def _():
        xpv = xp_ref[...]
        xx = jnp.sum(xpv * xpv, axis=0, keepdims=True)
        lane = lax.broadcasted_iota(jnp.int32, (1, NPAD), 1)
        xx_ref[...] = jnp.where(lane >= NPTS, jnp.float32(jnp.inf), xx)

    xt = xtp_ref[...]
    dot = jnp.dot(xt, xp_ref[...], preferred_element_type=jnp.float32)
    s0 = 2.0 * dot - xx_ref[...]
    col = lax.broadcasted_iota(jnp.int32, (RB, NPAD), 1)
    lane16 = lax.broadcasted_iota(jnp.int32, (RB, KNN), 1)
    neg = jnp.float32(-jnp.inf)

    m0 = jnp.max(s0, axis=1, keepdims=True)

    def kbody(k, carry):
        m, acc = carry
        idc = jnp.where(s0 == m, col, NPAD)
        j = jnp.min(idc, axis=1, keepdims=True)
        acc = jnp.where(lane16 == k, j, acc)
        m = jnp.max(jnp.where(s0 < m, s0, neg), axis=1, keepdims=True)
        return m, acc

    _, idxacc = lax.fori_loop(
        0, KNN, kbody, (m0, jnp.zeros((RB, KNN), jnp.int32)), unroll=8)
    idx_ref[...] = idxacc
    ut_ref[...] = jnp.dot(xt, w1t_ref[...], preferred_element_type=jnp.float32)
    wt_ref[...] = jnp.dot(xt, wdt_ref[...], preferred_element_type=jnp.float32)


_topk_call = pl.pallas_call(
    _topk_body,
    grid=(NSTEP,),
    in_specs=[
        pl.BlockSpec((CH, NPAD), lambda i: (0, 0)),
        pl.BlockSpec((RB, CH), lambda i: (i, 0)),
        pl.BlockSpec((CH, CH), lambda i: (0, 0)),
        pl.BlockSpec((CH, CH), lambda i: (0, 0)),
    ],
    out_specs=[
        pl.BlockSpec((RB, KNN), lambda i: (i, 0)),
        pl.BlockSpec((RB, CH), lambda i: (i, 0)),
        pl.BlockSpec((RB, CH), lambda i: (i, 0)),
    ],
    out_shape=[
        jax.ShapeDtypeStruct((NPAD, KNN), jnp.int32),
        jax.ShapeDtypeStruct((NPAD, CH), jnp.float32),
        jax.ShapeDtypeStruct((NPAD, CH), jnp.float32),
    ],
    scratch_shapes=[pltpu.VMEM((1, NPAD), jnp.float32)],
)


def _sc_body(ut_hbm, idx_hbm, s1_hbm, s2_hbm, m_hbm,
             idx_v, g_v, s1_v, s2_v, m_v, sem):
    wid = lax.axis_index("s") * 2 + lax.axis_index("c")

    def chunk_body(ci, carry):
        rowbase = wid * (PTS_PER_W * KNN // 128) + ci * IDX_SB
        pltpu.sync_copy(idx_hbm.at[pl.ds(rowbase, IDX_SB)], idx_v)
        for h in range(2):
            cps = [
                pltpu.async_copy(ut_hbm.at[idx_v.at[h * 4 + sb]],
                                 g_v.at[pl.ds(sb * 128, 128)], sem)
                for sb in range(4)
            ]
            for cp in cps:
                cp.wait()

            def pt_body(p, c2):
                r0 = p * KNN
                for c in range(4):
                    cs = pl.ds(c * 16, 16)
                    v = g_v[r0, cs]
                    s1 = v
                    s2 = v * v
                    mx = v
                    for r in range(1, KNN):
                        vv = g_v[r0 + r, cs]
                        s1 = s1 + vv
                        s2 = s2 + vv * vv
                        mx = jnp.maximum(mx, vv)
                    s1_v[p, cs] = s1
                    s2_v[p, cs] = s2
                    m_v[p, cs] = mx
                return c2

            lax.fori_loop(0, CHUNK_PTS // 2, pt_body, 0)
            pt0 = wid * PTS_PER_W + ci * CHUNK_PTS + h * (CHUNK_PTS // 2)
            pltpu.sync_copy(s1_v, s1_hbm.at[pl.ds(pt0, CHUNK_PTS // 2)])
            pltpu.sync_copy(s2_v, s2_hbm.at[pl.ds(pt0, CHUNK_PTS // 2)])
            pltpu.sync_copy(m_v, m_hbm.at[pl.ds(pt0, CHUNK_PTS // 2)])
        return carry

    lax.fori_loop(0, NCHUNK, chunk_body, 0)


@functools.lru_cache(maxsize=1)
def _get_sc_call():
    return functools.partial(
        pl.kernel,
        mesh=plsc.VectorSubcoreMesh(core_axis_name="c", subcore_axis_name="s"),
        out_type=[jax.ShapeDtypeStruct((NPAD, CH), jnp.float32)] * 3,
        scratch_types=[
            pltpu.VMEM((IDX_SB, 128), jnp.int32),
            pltpu.VMEM((CHUNK_ROWS // 2, 128), jnp.float32),
            pltpu.VMEM((CHUNK_PTS // 2, CH), jnp.float32),
            pltpu.VMEM((CHUNK_PTS // 2, CH), jnp.float32),
            pltpu.VMEM((CHUNK_PTS // 2, CH), jnp.float32),
            pltpu.SemaphoreType.DMA,
        ],
    )(_sc_body)


def _c1_body(s1_ref, s2_ref, wt_ref, acc_ref):
    i = pl.program_id(0)
    s1 = s1_ref[...]
    s2 = s2_ref[...]
    wt = wt_ref[...]
    p = jnp.concatenate([
        jnp.sum(s1, axis=0, keepdims=True),
        jnp.sum(s2, axis=0, keepdims=True),
        jnp.sum(wt * s1, axis=0, keepdims=True),
        jnp.sum(wt, axis=0, keepdims=True),
        jnp.sum(wt * wt, axis=0, keepdims=True),
        jnp.zeros((3, CH), jnp.float32),
    ], axis=0)

    @pl.when(i == 0)
    def _():
        acc_ref[...] = p

    @pl.when(i > 0)
    def _():
        acc_ref[...] = acc_ref[...] + p


_c1_call = pl.pallas_call(
    _c1_body,
    grid=(NPTS // CB1,),
    in_specs=[
        pl.BlockSpec((CB1, CH), lambda i: (i, 0)),
        pl.BlockSpec((CB1, CH), lambda i: (i, 0)),
        pl.BlockSpec((CB1, CH), lambda i: (i, 0)),
    ],
    out_specs=pl.BlockSpec((8, CH), lambda i: (0, 0)),
    out_shape=jax.ShapeDtypeStruct((8, CH), jnp.float32),
)


def _c2_body(m_ref, wt_ref, sums_ref, gam_ref, bet_ref, out_ref):
    sums = sums_ref[...]
    cnt = jnp.float32(NPTS * KNN)
    mean = (sums[0:1] + KNN * sums[3:4]) / cnt
    ey2 = (sums[1:2] + 2.0 * sums[2:3] + KNN * sums[4:5]) / cnt
    var = ey2 - mean * mean
    rstd = lax.rsqrt(var + 1e-5)
    z = (m_ref[...] + wt_ref[...] - mean) * (rstd * gam_ref[...]) + bet_ref[...]
    act = jnp.where(z > 0, z, 0.2 * z)
    out_ref[...] = act.T


_c2_call = pl.pallas_call(
    _c2_body,
    grid=(NPAD // CB2,),
    in_specs=[
        pl.BlockSpec((CB2, CH), lambda i: (i, 0)),
        pl.BlockSpec((CB2, CH), lambda i: (i, 0)),
        pl.BlockSpec((8, CH), lambda i: (0, 0)),
        pl.BlockSpec((1, CH), lambda i: (0, 0)),
        pl.BlockSpec((1, CH), lambda i: (0, 0)),
    ],
    out_specs=pl.BlockSpec((CH, CB2), lambda i: (0, i)),
    out_shape=jax.ShapeDtypeStruct((CH, NPAD), jnp.float32),
)


def kernel(x, W, gamma, beta):
    _, c, n = x.shape
    x2 = x[0]
    xp = jnp.pad(x2, ((0, 0), (0, NPAD - n)))
    xtp = xp.T
    w1t = W[:, :c].T
    wdt = (W[:, c:] - W[:, :c]).T

    idx, ut, wt = _topk_call(xp, xtp, w1t, wdt)
    idx2 = idx.reshape(-1, 128)
    utp = jnp.pad(ut, ((0, 0), (0, 128 - CH)))
    s1, s2, m = _get_sc_call()(utp, idx2)

    sums = _c1_call(s1, s2, wt)
    out = _c2_call(m, wt, sums, gamma.reshape(1, -1), beta.reshape(1, -1))
    return out[None, :, :n]


# speedup vs baseline: 1.0063x; 1.0062x over previous
import functools

import jax
import jax.numpy as jnp
from jax import lax
from jax.experimental import pallas as pl
from jax.experimental.pallas import tpu as pltpu
from jax.experimental.pallas import tpu_sc as plsc

KNN = 16
NPTS = 10000
CH = 64
NPAD = 10240
RB = 512
NSTEP = NPAD // RB

NWORK = 32
PTS_PER_W = NPAD // NWORK
CHUNK_PTS = 64
NCHUNK = PTS_PER_W // CHUNK_PTS
CHUNK_ROWS = CHUNK_PTS * KNN
IDX_SB = CHUNK_ROWS // 128

CB1 = 400
CB2 = 512


def _topk_body(xp_ref, xtp_ref, w1t_ref, wdt_ref, idx_ref, ut_ref, wt_ref,
               xx_ref):
    i = pl.program_id(0)

    @pl.when(i == 0)
    def _():
        xpv = xp_ref[...]
        xx = jnp.sum(xpv * xpv, axis=0, keepdims=True)
        lane = lax.broadcasted_iota(jnp.int32, (1, NPAD), 1)
        xx_ref[...] = jnp.where(lane >= NPTS, jnp.float32(jnp.inf), xx)

    xt = xtp_ref[...]
    dot = jnp.dot(xt, xp_ref[...], preferred_element_type=jnp.float32)
    s0 = 2.0 * dot - xx_ref[...]
    col = lax.broadcasted_iota(jnp.int32, (RB, NPAD), 1)
    lane16 = lax.broadcasted_iota(jnp.int32, (RB, KNN), 1)
    neg = jnp.float32(-jnp.inf)

    m0 = jnp.max(s0, axis=1, keepdims=True)

    def kbody(k, carry):
        m, acc = carry
        idc = jnp.where(s0 == m, col, NPAD)
        j = jnp.min(idc, axis=1, keepdims=True)
        acc = jnp.where(lane16 == k, j, acc)
        m = jnp.max(jnp.where(s0 < m, s0, neg), axis=1, keepdims=True)
        return m, acc

    _, idxacc = lax.fori_loop(
        0, KNN, kbody, (m0, jnp.zeros((RB, KNN), jnp.int32)), unroll=4)
    idx_ref[...] = idxacc
    ut_ref[...] = jnp.dot(xt, w1t_ref[...], preferred_element_type=jnp.float32)
    wt_ref[...] = jnp.dot(xt, wdt_ref[...], preferred_element_type=jnp.float32)


_topk_call = pl.pallas_call(
    _topk_body,
    grid=(NSTEP,),
    in_specs=[
        pl.BlockSpec((CH, NPAD), lambda i: (0, 0)),
        pl.BlockSpec((RB, CH), lambda i: (i, 0)),
        pl.BlockSpec((CH, CH), lambda i: (0, 0)),
        pl.BlockSpec((CH, CH), lambda i: (0, 0)),
    ],
    out_specs=[
        pl.BlockSpec((RB, KNN), lambda i: (i, 0)),
        pl.BlockSpec((RB, CH), lambda i: (i, 0)),
        pl.BlockSpec((RB, CH), lambda i: (i, 0)),
    ],
    out_shape=[
        jax.ShapeDtypeStruct((NPAD, KNN), jnp.int32),
        jax.ShapeDtypeStruct((NPAD, CH), jnp.float32),
        jax.ShapeDtypeStruct((NPAD, CH), jnp.float32),
    ],
    scratch_shapes=[pltpu.VMEM((1, NPAD), jnp.float32)],
)


def _sc_body(ut_hbm, idx_hbm, s1_hbm, s2_hbm, m_hbm,
             idx_v, g_v, s1_v, s2_v, m_v, sem):
    wid = lax.axis_index("s") * 2 + lax.axis_index("c")

    def chunk_body(ci, carry):
        rowbase = wid * (PTS_PER_W * KNN // 128) + ci * IDX_SB
        pltpu.sync_copy(idx_hbm.at[pl.ds(rowbase, IDX_SB)], idx_v)
        for h in range(2):
            cps = [
                pltpu.async_copy(ut_hbm.at[idx_v.at[h * 4 + sb]],
                                 g_v.at[pl.ds(sb * 128, 128)], sem)
                for sb in range(4)
            ]
            for cp in cps:
                cp.wait()

            def pt_body(p, c2):
                r0 = p * KNN
                for c in range(4):
                    cs = pl.ds(c * 16, 16)
                    v = g_v[r0, cs]
                    s1 = v
                    s2 = v * v
                    mx = v
                    for r in range(1, KNN):
                        vv = g_v[r0 + r, cs]
                        s1 = s1 + vv
                        s2 = s2 + vv * vv
                        mx = jnp.maximum(mx, vv)
                    s1_v[p, cs] = s1
                    s2_v[p, cs] = s2
                    m_v[p, cs] = mx
                return c2

            lax.fori_loop(0, CHUNK_PTS // 2, pt_body, 0)
            pt0 = wid * PTS_PER_W + ci * CHUNK_PTS + h * (CHUNK_PTS // 2)
            pltpu.sync_copy(s1_v, s1_hbm.at[pl.ds(pt0, CHUNK_PTS // 2)])
            pltpu.sync_copy(s2_v, s2_hbm.at[pl.ds(pt0, CHUNK_PTS // 2)])
            pltpu.sync_copy(m_v, m_hbm.at[pl.ds(pt0, CHUNK_PTS // 2)])
        return carry

    lax.fori_loop(0, NCHUNK, chunk_body, 0)


@functools.lru_cache(maxsize=1)
def _get_sc_call():
    return functools.partial(
        pl.kernel,
        mesh=plsc.VectorSubcoreMesh(core_axis_name="c", subcore_axis_name="s"),
        out_type=[jax.ShapeDtypeStruct((NPAD, CH), jnp.float32)] * 3,
        scratch_types=[
            pltpu.VMEM((IDX_SB, 128), jnp.int32),
            pltpu.VMEM((CHUNK_ROWS // 2, 128), jnp.float32),
            pltpu.VMEM((CHUNK_PTS // 2, CH), jnp.float32),
            pltpu.VMEM((CHUNK_PTS // 2, CH), jnp.float32),
            pltpu.VMEM((CHUNK_PTS // 2, CH), jnp.float32),
            pltpu.SemaphoreType.DMA,
        ],
    )(_sc_body)


def _c1_body(s1_ref, s2_ref, wt_ref, acc_ref):
    i = pl.program_id(0)
    s1 = s1_ref[...]
    s2 = s2_ref[...]
    wt = wt_ref[...]
    p = jnp.concatenate([
        jnp.sum(s1, axis=0, keepdims=True),
        jnp.sum(s2, axis=0, keepdims=True),
        jnp.sum(wt * s1, axis=0, keepdims=True),
        jnp.sum(wt, axis=0, keepdims=True),
        jnp.sum(wt * wt, axis=0, keepdims=True),
        jnp.zeros((3, CH), jnp.float32),
    ], axis=0)

    @pl.when(i == 0)
    def _():
        acc_ref[...] = p

    @pl.when(i > 0)
    def _():
        acc_ref[...] = acc_ref[...] + p


_c1_call = pl.pallas_call(
    _c1_body,
    grid=(NPTS // CB1,),
    in_specs=[
        pl.BlockSpec((CB1, CH), lambda i: (i, 0)),
        pl.BlockSpec((CB1, CH), lambda i: (i, 0)),
        pl.BlockSpec((CB1, CH), lambda i: (i, 0)),
    ],
    out_specs=pl.BlockSpec((8, CH), lambda i: (0, 0)),
    out_shape=jax.ShapeDtypeStruct((8, CH), jnp.float32),
)


def _c2_body(m_ref, wt_ref, sums_ref, gam_ref, bet_ref, out_ref):
    sums = sums_ref[...]
    cnt = jnp.float32(NPTS * KNN)
    mean = (sums[0:1] + KNN * sums[3:4]) / cnt
    ey2 = (sums[1:2] + 2.0 * sums[2:3] + KNN * sums[4:5]) / cnt
    var = ey2 - mean * mean
    rstd = lax.rsqrt(var + 1e-5)
    z = (m_ref[...] + wt_ref[...] - mean) * (rstd * gam_ref[...]) + bet_ref[...]
    act = jnp.where(z > 0, z, 0.2 * z)
    out_ref[...] = act.T


_c2_call = pl.pallas_call(
    _c2_body,
    grid=(NPAD // CB2,),
    in_specs=[
        pl.BlockSpec((CB2, CH), lambda i: (i, 0)),
        pl.BlockSpec((CB2, CH), lambda i: (i, 0)),
        pl.BlockSpec((8, CH), lambda i: (0, 0)),
        pl.BlockSpec((1, CH), lambda i: (0, 0)),
        pl.BlockSpec((1, CH), lambda i: (0, 0)),
    ],
    out_specs=pl.BlockSpec((CH, CB2), lambda i: (0, i)),
    out_shape=jax.ShapeDtypeStruct((CH, NPAD), jnp.float32),
)


def kernel(x, W, gamma, beta):
    _, c, n = x.shape
    x2 = x[0]
    xp = jnp.pad(x2, ((0, 0), (0, NPAD - n)))
    xtp = xp.T
    w1t = W[:, :c].T
    wdt = (W[:, c:] - W[:, :c]).T

    idx, ut, wt = _topk_call(xp, xtp, w1t, wdt)
    idx2 = idx.reshape(-1, 128)
    utp = jnp.pad(ut, ((0, 0), (0, 128 - CH)))
    s1, s2, m = _get_sc_call()(utp, idx2)

    sums = _c1_call(s1, s2, wt)
    out = _c2_call(m, wt, sums, gamma.reshape(1, -1), beta.reshape(1, -1))
    return out[None, :, :n]


# speedup vs baseline: 1.1453x; 1.1382x over previous
import functools

import jax
import jax.numpy as jnp
from jax import lax
from jax.experimental import pallas as pl
from jax.experimental.pallas import tpu as pltpu
from jax.experimental.pallas import tpu_sc as plsc

KNN = 16
NPTS = 10000
CH = 64
NPAD = 10240
RB = 512
NSTEP = NPAD // RB

NWORK = 32
PTS_PER_W = NPAD // NWORK
CHUNK_PTS = 64
NCHUNK = PTS_PER_W // CHUNK_PTS
CHUNK_ROWS = CHUNK_PTS * KNN
IDX_SB = CHUNK_ROWS // 128

CB1 = 400
CB2 = 512


def _topk_body(xp_ref, xtp_ref, w1t_ref, wdt_ref, idx_ref, ut_ref, wt_ref,
               xx_ref):
    i = pl.program_id(0)

    @pl.when(i == 0)
    def _():
        xpv = xp_ref[...]
        xx = jnp.sum(xpv * xpv, axis=0, keepdims=True)
        lane = lax.broadcasted_iota(jnp.int32, (1, NPAD), 1)
        xx_ref[...] = jnp.where(lane >= NPTS, jnp.float32(jnp.inf), xx)

    xt = xtp_ref[...]
    dot = jnp.dot(xt, xp_ref[...], preferred_element_type=jnp.float32)
    s0 = 2.0 * dot - xx_ref[...]
    colf = lax.broadcasted_iota(jnp.int32, (RB, NPAD), 1).astype(jnp.float32)
    lane16 = lax.broadcasted_iota(jnp.int32, (RB, KNN), 1)
    neg = jnp.float32(-jnp.inf)
    big = jnp.float32(NPAD)

    m0 = jnp.max(s0, axis=1, keepdims=True)

    def kbody(k, carry):
        m, acc = carry
        idc = jnp.where(s0 == m, colf, big)
        j = jnp.min(idc, axis=1, keepdims=True).astype(jnp.int32)
        acc = jnp.where(lane16 == k, j, acc)
        m = jnp.max(jnp.where(s0 < m, s0, neg), axis=1, keepdims=True)
        return m, acc

    _, idxacc = lax.fori_loop(
        0, KNN, kbody, (m0, jnp.zeros((RB, KNN), jnp.int32)), unroll=4)
    idx_ref[...] = idxacc
    ut_ref[...] = jnp.dot(xt, w1t_ref[...], preferred_element_type=jnp.float32)
    wt_ref[...] = jnp.dot(xt, wdt_ref[...], preferred_element_type=jnp.float32)


_topk_call = pl.pallas_call(
    _topk_body,
    grid=(NSTEP,),
    in_specs=[
        pl.BlockSpec((CH, NPAD), lambda i: (0, 0)),
        pl.BlockSpec((RB, CH), lambda i: (i, 0)),
        pl.BlockSpec((CH, CH), lambda i: (0, 0)),
        pl.BlockSpec((CH, CH), lambda i: (0, 0)),
    ],
    out_specs=[
        pl.BlockSpec((RB, KNN), lambda i: (i, 0)),
        pl.BlockSpec((RB, CH), lambda i: (i, 0)),
        pl.BlockSpec((RB, CH), lambda i: (i, 0)),
    ],
    out_shape=[
        jax.ShapeDtypeStruct((NPAD, KNN), jnp.int32),
        jax.ShapeDtypeStruct((NPAD, CH), jnp.float32),
        jax.ShapeDtypeStruct((NPAD, CH), jnp.float32),
    ],
    scratch_shapes=[pltpu.VMEM((1, NPAD), jnp.float32)],
)


def _sc_body(ut_hbm, idx_hbm, s1_hbm, s2_hbm, m_hbm,
             idx_v, g_v, s1_v, s2_v, m_v, sem):
    wid = lax.axis_index("s") * 2 + lax.axis_index("c")

    def chunk_body(ci, carry):
        rowbase = wid * (PTS_PER_W * KNN // 128) + ci * IDX_SB
        pltpu.sync_copy(idx_hbm.at[pl.ds(rowbase, IDX_SB)], idx_v)
        for h in range(2):
            cps = [
                pltpu.async_copy(ut_hbm.at[idx_v.at[h * 4 + sb]],
                                 g_v.at[pl.ds(sb * 128, 128)], sem)
                for sb in range(4)
            ]
            for cp in cps:
                cp.wait()

            def pt_body(p, c2):
                r0 = p * KNN
                for c in range(4):
                    cs = pl.ds(c * 16, 16)
                    v = g_v[r0, cs]
                    s1 = v
                    s2 = v * v
                    mx = v
                    for r in range(1, KNN):
                        vv = g_v[r0 + r, cs]
                        s1 = s1 + vv
                        s2 = s2 + vv * vv
                        mx = jnp.maximum(mx, vv)
                    s1_v[p, cs] = s1
                    s2_v[p, cs] = s2
                    m_v[p, cs] = mx
                return c2

            lax.fori_loop(0, CHUNK_PTS // 2, pt_body, 0)
            pt0 = wid * PTS_PER_W + ci * CHUNK_PTS + h * (CHUNK_PTS // 2)
            pltpu.sync_copy(s1_v, s1_hbm.at[pl.ds(pt0, CHUNK_PTS // 2)])
            pltpu.sync_copy(s2_v, s2_hbm.at[pl.ds(pt0, CHUNK_PTS // 2)])
            pltpu.sync_copy(m_v, m_hbm.at[pl.ds(pt0, CHUNK_PTS // 2)])
        return carry

    lax.fori_loop(0, NCHUNK, chunk_body, 0)


@functools.lru_cache(maxsize=1)
def _get_sc_call():
    return functools.partial(
        pl.kernel,
        mesh=plsc.VectorSubcoreMesh(core_axis_name="c", subcore_axis_name="s"),
        out_type=[jax.ShapeDtypeStruct((NPAD, CH), jnp.float32)] * 3,
        scratch_types=[
            pltpu.VMEM((IDX_SB, 128), jnp.int32),
            pltpu.VMEM((CHUNK_ROWS // 2, 128), jnp.float32),
            pltpu.VMEM((CHUNK_PTS // 2, CH), jnp.float32),
            pltpu.VMEM((CHUNK_PTS // 2, CH), jnp.float32),
            pltpu.VMEM((CHUNK_PTS // 2, CH), jnp.float32),
            pltpu.SemaphoreType.DMA,
        ],
    )(_sc_body)


def _c1_body(s1_ref, s2_ref, wt_ref, acc_ref):
    i = pl.program_id(0)
    s1 = s1_ref[...]
    s2 = s2_ref[...]
    wt = wt_ref[...]
    p = jnp.concatenate([
        jnp.sum(s1, axis=0, keepdims=True),
        jnp.sum(s2, axis=0, keepdims=True),
        jnp.sum(wt * s1, axis=0, keepdims=True),
        jnp.sum(wt, axis=0, keepdims=True),
        jnp.sum(wt * wt, axis=0, keepdims=True),
        jnp.zeros((3, CH), jnp.float32),
    ], axis=0)

    @pl.when(i == 0)
    def _():
        acc_ref[...] = p

    @pl.when(i > 0)
    def _():
        acc_ref[...] = acc_ref[...] + p


_c1_call = pl.pallas_call(
    _c1_body,
    grid=(NPTS // CB1,),
    in_specs=[
        pl.BlockSpec((CB1, CH), lambda i: (i, 0)),
        pl.BlockSpec((CB1, CH), lambda i: (i, 0)),
        pl.BlockSpec((CB1, CH), lambda i: (i, 0)),
    ],
    out_specs=pl.BlockSpec((8, CH), lambda i: (0, 0)),
    out_shape=jax.ShapeDtypeStruct((8, CH), jnp.float32),
)


def _c2_body(m_ref, wt_ref, sums_ref, gam_ref, bet_ref, out_ref):
    sums = sums_ref[...]
    cnt = jnp.float32(NPTS * KNN)
    mean = (sums[0:1] + KNN * sums[3:4]) / cnt
    ey2 = (sums[1:2] + 2.0 * sums[2:3] + KNN * sums[4:5]) / cnt
    var = ey2 - mean * mean
    rstd = lax.rsqrt(var + 1e-5)
    z = (m_ref[...] + wt_ref[...] - mean) * (rstd * gam_ref[...]) + bet_ref[...]
    act = jnp.where(z > 0, z, 0.2 * z)
    out_ref[...] = act.T


_c2_call = pl.pallas_call(
    _c2_body,
    grid=(NPAD // CB2,),
    in_specs=[
        pl.BlockSpec((CB2, CH), lambda i: (i, 0)),
        pl.BlockSpec((CB2, CH), lambda i: (i, 0)),
        pl.BlockSpec((8, CH), lambda i: (0, 0)),
        pl.BlockSpec((1, CH), lambda i: (0, 0)),
        pl.BlockSpec((1, CH), lambda i: (0, 0)),
    ],
    out_specs=pl.BlockSpec((CH, CB2), lambda i: (0, i)),
    out_shape=jax.ShapeDtypeStruct((CH, NPAD), jnp.float32),
)


def kernel(x, W, gamma, beta):
    _, c, n = x.shape
    x2 = x[0]
    xp = jnp.pad(x2, ((0, 0), (0, NPAD - n)))
    xtp = xp.T
    w1t = W[:, :c].T
    wdt = (W[:, c:] - W[:, :c]).T

    idx, ut, wt = _topk_call(xp, xtp, w1t, wdt)
    idx2 = idx.reshape(-1, 128)
    utp = jnp.pad(ut, ((0, 0), (0, 128 - CH)))
    s1, s2, m = _get_sc_call()(utp, idx2)

    sums = _c1_call(s1, s2, wt)
    out = _c2_call(m, wt, sums, gamma.reshape(1, -1), beta.reshape(1, -1))
    return out[None, :, :n]
